# baseline (speedup 1.0000x reference)
import functools

import jax
import jax.numpy as jnp
from jax.experimental import pallas as pl
from jax.experimental.pallas import tpu as pltpu

_NEG_INF = float("-inf")


def _att_pool_kernel(x_ref, br_ref, wt_ref, b_ref, out_ref,
                     m_ref, s_ref, acc_ref, *, nblocks, B, subblocks):
    k = pl.program_id(0)
    R = x_ref.shape[0]

    @pl.when(k == 0)
    def _init():
        m_ref[...] = jnp.full_like(m_ref, _NEG_INF)
        s_ref[...] = jnp.zeros_like(s_ref)
        acc_ref[...] = jnp.zeros_like(acc_ref)

    Rh = R // subblocks
    for h in range(subblocks):
        xb_bf = x_ref[h * Rh:(h + 1) * Rh, :].astype(jnp.bfloat16)
        bi_row = br_ref[0][:, h * Rh:(h + 1) * Rh].astype(jnp.int16)

        oh = bi_row == jax.lax.broadcasted_iota(jnp.int16, (B, Rh), 0)

        g_row = jax.lax.dot_general(
            wt_ref[...].astype(jnp.bfloat16), xb_bf,
            (((1,), (1,)), ((), ())),
            preferred_element_type=jnp.float32)
        g_row = g_row + b_ref[...]
        g_bf = g_row.astype(jnp.bfloat16)

        masked = jnp.where(oh, g_bf, jnp.bfloat16(_NEG_INF))
        bmax = jnp.max(masked, axis=1, keepdims=True)
        m_old = m_ref[...]
        m_new = jnp.maximum(m_old, bmax.astype(jnp.float32))
        m_ref[...] = m_new
        scale = jnp.where(m_old == _NEG_INF, 0.0,
                          jnp.exp(m_old - m_new))

        m_bf = m_new.astype(jnp.bfloat16)
        m_g_row = jnp.sum(jnp.where(oh, m_bf, jnp.bfloat16(0.0)),
                          axis=0, keepdims=True)
        p_row = jnp.exp(g_row - m_g_row.astype(jnp.float32))
        weights = jnp.where(oh, p_row.astype(jnp.bfloat16),
                            jnp.bfloat16(0.0))

        ones = jnp.ones((Rh, 1), jnp.bfloat16)
        s_upd = jax.lax.dot_general(
            weights, ones, (((1,), (0,)), ((), ())),
            preferred_element_type=jnp.float32)
        s_ref[...] = s_ref[...] * scale + s_upd

        upd = jax.lax.dot_general(
            weights, xb_bf, (((1,), (0,)), ((), ())),
            preferred_element_type=jnp.float32)
        acc_ref[...] = acc_ref[...] * scale + upd

    @pl.when(k == nblocks - 1)
    def _finish():
        s = s_ref[...]
        out_ref[...] = jnp.where(s > 0.0, acc_ref[...] / s, 0.0)


@functools.partial(jax.jit, static_argnames=("block_rows", "subblocks"))
def _att_pool(x, batch_i32, Wt, b2, *, block_rows, subblocks):
    N, D = x.shape
    B = 128
    R = block_rows
    npad = (-N) % R
    if npad:
        x = jnp.concatenate([x, jnp.zeros((npad, D), x.dtype)], axis=0)
        batch_i32 = jnp.concatenate(
            [batch_i32, jnp.full((npad,), B, jnp.int32)], axis=0)
    nb = (N + npad) // R
    br = batch_i32.reshape(nb, 1, R)

    grid = (nb,)
    kernel_fn = functools.partial(_att_pool_kernel, nblocks=nb, B=B,
                                  subblocks=subblocks)
    return pl.pallas_call(
        kernel_fn,
        grid=grid,
        in_specs=[
            pl.BlockSpec((R, D), lambda k: (k, 0)),
            pl.BlockSpec((1, 1, R), lambda k: (k, 0, 0)),
            pl.BlockSpec((1, D), lambda k: (0, 0)),
            pl.BlockSpec((1, 1), lambda k: (0, 0)),
        ],
        out_specs=pl.BlockSpec((B, D), lambda k: (0, 0)),
        out_shape=jax.ShapeDtypeStruct((B, D), jnp.float32),
        scratch_shapes=[
            pltpu.VMEM((B, 1), jnp.float32),
            pltpu.VMEM((B, 1), jnp.float32),
            pltpu.VMEM((B, D), jnp.float32),
        ],
        compiler_params=pltpu.CompilerParams(
            dimension_semantics=("arbitrary",),
        ),
    )(x, br, Wt, b2)


def kernel(x, batch, W, b):
    batch_i32 = batch.astype(jnp.int32)
    Wt = W.reshape(1, -1)
    b2 = b.reshape(1, 1).astype(jnp.float32)
    return _att_pool(x, batch_i32, Wt, b2, block_rows=10000, subblocks=1)


# speedup vs baseline: 1.0262x; 1.0262x over previous
import functools

import jax
import jax.numpy as jnp
from jax.experimental import pallas as pl
from jax.experimental.pallas import tpu as pltpu

_NEG_INF = float("-inf")
_NBUF = 3


def _att_pool_kernel(x_hbm, br_ref, wt_ref, b_ref, out_ref,
                     xbuf, sem, m_ref, s_ref, acc_ref, *, nblocks, B, R):
    k = pl.program_id(0)

    def _copy(c, slot):
        return pltpu.make_async_copy(
            x_hbm.at[pl.ds(c * R, R), :], xbuf.at[slot], sem.at[slot])

    @pl.when(k == 0)
    def _init():
        m_ref[...] = jnp.full_like(m_ref, _NEG_INF)
        s_ref[...] = jnp.zeros_like(s_ref)
        acc_ref[...] = jnp.zeros_like(acc_ref)
        _copy(0, 0).start()
        if nblocks > 1:
            _copy(1, 1).start()

    slot = jax.lax.rem(k, _NBUF)
    _copy(k, slot).wait()

    @pl.when(k + 2 < nblocks)
    def _prefetch():
        nxt = k + 2
        _copy(nxt, jax.lax.rem(nxt, _NBUF)).start()

    xb_bf = xbuf[slot].astype(jnp.bfloat16)
    bi_row = br_ref[0].astype(jnp.int16)

    oh = bi_row == jax.lax.broadcasted_iota(jnp.int16, (B, R), 0)

    g_row = jax.lax.dot_general(
        wt_ref[...].astype(jnp.bfloat16), xb_bf,
        (((1,), (1,)), ((), ())),
        preferred_element_type=jnp.float32)
    g_row = g_row + b_ref[...]
    g_bf = g_row.astype(jnp.bfloat16)

    masked = jnp.where(oh, g_bf, jnp.bfloat16(_NEG_INF))
    bmax = jnp.max(masked, axis=1, keepdims=True)
    m_old = m_ref[...]
    m_new = jnp.maximum(m_old, bmax.astype(jnp.float32))
    m_ref[...] = m_new
    scale = jnp.where(m_old == _NEG_INF, 0.0,
                      jnp.exp(m_old - m_new))

    m_bf = m_new.astype(jnp.bfloat16)
    m_g_row = jnp.sum(jnp.where(oh, m_bf, jnp.bfloat16(0.0)),
                      axis=0, keepdims=True)
    p_row = jnp.exp(g_row - m_g_row.astype(jnp.float32))
    weights = jnp.where(oh, p_row.astype(jnp.bfloat16),
                        jnp.bfloat16(0.0))

    ones = jnp.ones((R, 1), jnp.bfloat16)
    s_upd = jax.lax.dot_general(
        weights, ones, (((1,), (0,)), ((), ())),
        preferred_element_type=jnp.float32)
    s_ref[...] = s_ref[...] * scale + s_upd

    upd = jax.lax.dot_general(
        weights, xb_bf, (((1,), (0,)), ((), ())),
        preferred_element_type=jnp.float32)
    acc_ref[...] = acc_ref[...] * scale + upd

    @pl.when(k == nblocks - 1)
    def _finish():
        s = s_ref[...]
        out_ref[...] = jnp.where(s > 0.0, acc_ref[...] / s, 0.0)


@functools.partial(jax.jit, static_argnames=("block_rows",))
def _att_pool(x, batch_i32, Wt, b2, *, block_rows):
    N, D = x.shape
    B = 128
    R = block_rows
    npad = (-N) % R
    if npad:
        x = jnp.concatenate([x, jnp.zeros((npad, D), x.dtype)], axis=0)
        batch_i32 = jnp.concatenate(
            [batch_i32, jnp.full((npad,), B, jnp.int32)], axis=0)
    nb = (N + npad) // R
    br = batch_i32.reshape(nb, 1, R)

    grid = (nb,)
    kernel_fn = functools.partial(_att_pool_kernel, nblocks=nb, B=B, R=R)
    return pl.pallas_call(
        kernel_fn,
        grid=grid,
        in_specs=[
            pl.BlockSpec(memory_space=pltpu.MemorySpace.HBM),
            pl.BlockSpec((1, 1, R), lambda k: (k, 0, 0)),
            pl.BlockSpec((1, D), lambda k: (0, 0)),
            pl.BlockSpec((1, 1), lambda k: (0, 0)),
        ],
        out_specs=pl.BlockSpec((B, D), lambda k: (0, 0)),
        out_shape=jax.ShapeDtypeStruct((B, D), jnp.float32),
        scratch_shapes=[
            pltpu.VMEM((_NBUF, R, D), jnp.float32),
            pltpu.SemaphoreType.DMA((_NBUF,)),
            pltpu.VMEM((B, 1), jnp.float32),
            pltpu.VMEM((B, 1), jnp.float32),
            pltpu.VMEM((B, D), jnp.float32),
        ],
        compiler_params=pltpu.CompilerParams(
            dimension_semantics=("arbitrary",),
        ),
    )(x, br, Wt, b2)


def kernel(x, batch, W, b):
    batch_i32 = batch.astype(jnp.int32)
    Wt = W.reshape(1, -1)
    b2 = b.reshape(1, 1).astype(jnp.float32)
    return _att_pool(x, batch_i32, Wt, b2, block_rows=5000)
